# baseline (speedup 1.0000x reference)
import jax
import jax.numpy as jnp
from jax.experimental import pallas as pl


_BLOCK_ROWS = 2048
_CHUNK = 128


def _scan_kernel(x_ref, texc_ref, oexc_ref, o_ref):
    x = x_ref[...]
    rows, n = x.shape
    c = _CHUNK
    nchunk = n // c
    f32 = jnp.float32

    texc = texc_ref[...]
    oexc = oexc_ref[...]

    carry8 = jnp.dot(x, oexc, preferred_element_type=f32)
    for k in range(nchunk):
        part = jnp.dot(
            x[:, k * c : (k + 1) * c], texc, preferred_element_type=f32
        )
        ck = jnp.broadcast_to(carry8[:, k : k + 1], (rows, c))
        o_ref[:, k * c : (k + 1) * c] = part + ck


def kernel(x):
    n_rows, n = x.shape
    c = _CHUNK
    nchunk = n // c
    f32 = jnp.float32

    rr = jax.lax.broadcasted_iota(jnp.int32, (c, c), 0)
    cc = jax.lax.broadcasted_iota(jnp.int32, (c, c), 1)
    texc = (rr < cc).astype(f32)

    jr = jax.lax.broadcasted_iota(jnp.int32, (n, nchunk), 0) // c
    kc = jax.lax.broadcasted_iota(jnp.int32, (n, nchunk), 1)
    oexc = (jr < kc).astype(f32)

    grid = (n_rows // _BLOCK_ROWS,)
    return pl.pallas_call(
        _scan_kernel,
        grid=grid,
        in_specs=[
            pl.BlockSpec((_BLOCK_ROWS, n), lambda i: (i, 0)),
            pl.BlockSpec((c, c), lambda i: (0, 0)),
            pl.BlockSpec((n, nchunk), lambda i: (0, 0)),
        ],
        out_specs=pl.BlockSpec((_BLOCK_ROWS, n), lambda i: (i, 0)),
        out_shape=jax.ShapeDtypeStruct((n_rows, n), x.dtype),
    )(x, texc, oexc)


# speedup vs baseline: 1.1640x; 1.1640x over previous
import jax
import jax.numpy as jnp
from jax.experimental import pallas as pl


_BLOCK_ROWS = 2048
_CHUNK = 128


def _scan_kernel(x_ref, o_ref):
    x = x_ref[...]
    rows, n = x.shape
    c = _CHUNK
    nchunk = n // c
    f32 = jnp.float32

    rr = jax.lax.broadcasted_iota(jnp.int32, (c, c), 0)
    cc = jax.lax.broadcasted_iota(jnp.int32, (c, c), 1)
    texc = (rr < cc).astype(f32)

    carry = jnp.zeros((rows, 1), dtype=f32)
    for k in range(nchunk):
        xk = x[:, k * c : (k + 1) * c]
        part = jnp.dot(xk, texc, preferred_element_type=f32)
        o_ref[:, k * c : (k + 1) * c] = part + jnp.broadcast_to(carry, (rows, c))
        if k + 1 < nchunk:
            carry = carry + part[:, c - 1 : c] + xk[:, c - 1 : c]


def kernel(x):
    n_rows, n = x.shape
    grid = (n_rows // _BLOCK_ROWS,)
    return pl.pallas_call(
        _scan_kernel,
        grid=grid,
        in_specs=[pl.BlockSpec((_BLOCK_ROWS, n), lambda i: (i, 0))],
        out_specs=pl.BlockSpec((_BLOCK_ROWS, n), lambda i: (i, 0)),
        out_shape=jax.ShapeDtypeStruct((n_rows, n), x.dtype),
    )(x)
